# baseline (speedup 1.0000x reference)
import functools

import jax
import jax.numpy as jnp
from jax import lax
from jax.experimental import pallas as pl
from jax.experimental.pallas import tpu as pltpu
from jax.experimental.pallas import tpu_sc as plsc

_NUM_CLASSES = 100000
_D = 64
_B = 16384
_LAMBDA_C = 0.003

_INFO = plsc.get_sparse_core_info()
_NC, _NS, _L = _INFO.num_cores, _INFO.num_subcores, _INFO.num_lanes
_NW = _NC * _NS
_BPW = _B // _NW
_GCHUNK = 128
_NG = _BPW // _GCHUNK
_DV = _D // _L


@functools.partial(
    pl.kernel,
    out_type=jax.ShapeDtypeStruct((_NW, _L), jnp.float32),
    mesh=plsc.VectorSubcoreMesh(core_axis_name="c", subcore_axis_name="s"),
    scratch_types=[
        pltpu.VMEM((_NG, _GCHUNK), jnp.int32),
        pltpu.VMEM((_BPW, _D), jnp.float32),
        pltpu.VMEM((_BPW, _D), jnp.float32),
        pltpu.VMEM((_L,), jnp.float32),
        pltpu.SemaphoreType.DMA,
    ],
    compiler_params=pltpu.CompilerParams(use_tc_tiling_on_sc=False),
)
def _center_loss_sc(feat_hbm, idx_hbm, centers_hbm, out_hbm,
                    idx_v, rows_v, feats_v, part_v, sem):
    wid = lax.axis_index("s") * _NC + lax.axis_index("c")
    base = wid * _BPW

    pltpu.sync_copy(idx_hbm.at[wid], idx_v)
    gathers = [
        pltpu.async_copy(
            centers_hbm.at[idx_v.at[j]],
            rows_v.at[pl.ds(j * _GCHUNK, _GCHUNK)],
            sem,
        )
        for j in range(_NG)
    ]
    pltpu.sync_copy(feat_hbm.at[pl.ds(base, _BPW)], feats_v)
    for g in gathers:
        g.wait()

    def body(r, accs):
        out = []
        for c in range(_DV):
            d = feats_v[r, pl.ds(c * _L, _L)] - rows_v[r, pl.ds(c * _L, _L)]
            out.append(accs[c] + d * d)
        return tuple(out)

    zero = jnp.zeros((_L,), jnp.float32)
    accs = lax.fori_loop(0, _BPW, body, (zero,) * _DV)
    total = (accs[0] + accs[1]) + (accs[2] + accs[3])
    part_v[...] = total * (_LAMBDA_C / float(_B * _D))
    pltpu.sync_copy(part_v, out_hbm.at[wid])


def kernel(features, labels, centers):
    idx = labels.astype(jnp.int32).reshape(_NW, _NG, _GCHUNK)
    partials = _center_loss_sc(features, idx, centers)
    return jnp.sum(partials)
